# baseline (speedup 1.0000x reference)
import functools

import jax
import jax.numpy as jnp
from jax.experimental import pallas as pl
from jax.experimental.pallas import tpu as pltpu

NUM_EMBEDDINGS = 1024
EMBEDDING_DIM = 64
COMMITMENT_COST = 0.1

TILE = 1024


def _vq_kernel(z_ref, cb_ref, out_ref, sq_ref):
    i = pl.program_id(0)
    z = z_ref[...]
    cb = cb_ref[...]

    z_sq = jnp.sum(z * z, axis=1, keepdims=True)
    cb_sq = jnp.sum(cb * cb, axis=1)
    cross = jax.lax.dot_general(
        z, cb, dimension_numbers=(((1,), (1,)), ((), ())),
        preferred_element_type=jnp.float32)
    dist = (z_sq + cb_sq[None, :]) - 2.0 * cross

    min_d = jnp.min(dist, axis=1, keepdims=True)
    col = jax.lax.broadcasted_iota(jnp.int32, dist.shape, 1)
    idx = jnp.min(jnp.where(dist == min_d, col, NUM_EMBEDDINGS), axis=1,
                  keepdims=True)

    onehot = (col == idx).astype(jnp.float32)
    zq = jax.lax.dot_general(
        onehot, cb, dimension_numbers=(((1,), (0,)), ((), ())),
        preferred_element_type=jnp.float32,
        precision=jax.lax.Precision.HIGHEST)

    diff = zq - z
    out_ref[...] = z + diff
    part = jnp.sum(diff * diff)

    @pl.when(i == 0)
    def _():
        sq_ref[0, 0] = 0.0

    sq_ref[0, 0] += part


@jax.jit
def kernel(z, codebook):
    zz = z[0]
    n = zz.shape[0] * zz.shape[1]
    z_flat = zz.reshape(n, EMBEDDING_DIM)
    grid = n // TILE

    out, sqsum = pl.pallas_call(
        _vq_kernel,
        grid=(grid,),
        in_specs=[
            pl.BlockSpec((TILE, EMBEDDING_DIM), lambda i: (i, 0)),
            pl.BlockSpec((NUM_EMBEDDINGS, EMBEDDING_DIM), lambda i: (0, 0)),
        ],
        out_specs=[
            pl.BlockSpec((TILE, EMBEDDING_DIM), lambda i: (i, 0)),
            pl.BlockSpec((1, 1), lambda i: (0, 0), memory_space=pltpu.SMEM),
        ],
        out_shape=[
            jax.ShapeDtypeStruct((n, EMBEDDING_DIM), jnp.float32),
            jax.ShapeDtypeStruct((1, 1), jnp.float32),
        ],
    )(z_flat, codebook)

    mean_sq = sqsum[0, 0] / (n * EMBEDDING_DIM)
    vq_loss = mean_sq + COMMITMENT_COST * mean_sq
    return (out.reshape(zz.shape), vq_loss)


# speedup vs baseline: 1.6284x; 1.6284x over previous
import functools

import jax
import jax.numpy as jnp
from jax.experimental import pallas as pl
from jax.experimental.pallas import tpu as pltpu

NUM_EMBEDDINGS = 1024
EMBEDDING_DIM = 64
COMMITMENT_COST = 0.1

TILE = 1024


def _vq_kernel(z_ref, cb_ref, out_ref, sq_ref):
    i = pl.program_id(0)
    z = z_ref[...]
    cb = cb_ref[...]

    z_sq = jnp.sum(z * z, axis=1, keepdims=True)
    cb_sq = jnp.sum(cb * cb, axis=1)
    cross = jax.lax.dot_general(
        z, cb, dimension_numbers=(((1,), (1,)), ((), ())),
        preferred_element_type=jnp.float32)
    dist = (z_sq + cb_sq[None, :]) - 2.0 * cross

    min_d = jnp.min(dist, axis=1, keepdims=True)
    col = jax.lax.broadcasted_iota(jnp.int32, dist.shape, 1)
    idx = jnp.min(jnp.where(dist == min_d, col, NUM_EMBEDDINGS), axis=1,
                  keepdims=True)

    onehot = (col == idx).astype(jnp.float32)
    zq = jax.lax.dot_general(
        onehot, cb, dimension_numbers=(((1,), (0,)), ((), ())),
        preferred_element_type=jnp.float32)

    diff = zq - z
    out_ref[...] = z + diff
    part = jnp.sum(diff * diff)

    @pl.when(i == 0)
    def _():
        sq_ref[0, 0] = 0.0

    sq_ref[0, 0] += part


@jax.jit
def kernel(z, codebook):
    zz = z[0]
    n = zz.shape[0] * zz.shape[1]
    z_flat = zz.reshape(n, EMBEDDING_DIM)
    grid = n // TILE

    out, sqsum = pl.pallas_call(
        _vq_kernel,
        grid=(grid,),
        in_specs=[
            pl.BlockSpec((TILE, EMBEDDING_DIM), lambda i: (i, 0)),
            pl.BlockSpec((NUM_EMBEDDINGS, EMBEDDING_DIM), lambda i: (0, 0)),
        ],
        out_specs=[
            pl.BlockSpec((TILE, EMBEDDING_DIM), lambda i: (i, 0)),
            pl.BlockSpec((1, 1), lambda i: (0, 0), memory_space=pltpu.SMEM),
        ],
        out_shape=[
            jax.ShapeDtypeStruct((n, EMBEDDING_DIM), jnp.float32),
            jax.ShapeDtypeStruct((1, 1), jnp.float32),
        ],
    )(z_flat, codebook)

    mean_sq = sqsum[0, 0] / (n * EMBEDDING_DIM)
    vq_loss = mean_sq + COMMITMENT_COST * mean_sq
    return (out.reshape(zz.shape), vq_loss)
